# Initial kernel scaffold; baseline (speedup 1.0000x reference)
import jax
import jax.numpy as jnp
from jax.experimental import pallas as pl


def kernel(x, edge_index, edge_type, bases1, comp1, root1, bias1, gamma1, beta1, bases2, comp2, root2, bias2, gamma2, beta2, bases3, comp3, root3, bias3):
    raise NotImplementedError("write your pallas kernel here")


# speedup vs baseline: 1.0713x; 1.0713x over previous
import functools

import jax
import jax.numpy as jnp
from jax.experimental import pallas as pl


def _dense_block(agg_ref, inv_ref, x_ref, w_ref, root_ref, bias_ref, out_ref):
    acc = jnp.dot(x_ref[...], root_ref[...], preferred_element_type=jnp.float32)
    for r in range(8):
        h = agg_ref[r] * inv_ref[r]
        acc = acc + jnp.dot(h, w_ref[r], preferred_element_type=jnp.float32)
    out_ref[...] = acc + bias_ref[...]


def _dense(agg, inv, x, w, root, bias):
    n, d = x.shape
    B = 1000
    grid = (n // B,)
    return pl.pallas_call(
        _dense_block,
        grid=grid,
        in_specs=[
            pl.BlockSpec((8, B, d), lambda i: (0, i, 0)),
            pl.BlockSpec((8, B, 1), lambda i: (0, i, 0)),
            pl.BlockSpec((B, d), lambda i: (i, 0)),
            pl.BlockSpec((8, d, d), lambda i: (0, 0, 0)),
            pl.BlockSpec((d, d), lambda i: (0, 0)),
            pl.BlockSpec((1, d), lambda i: (0, 0)),
        ],
        out_specs=pl.BlockSpec((B, d), lambda i: (i, 0)),
        out_shape=jax.ShapeDtypeStruct((n, d), jnp.float32),
    )(agg, inv, x, w, root, bias)


def _conv(x, src, seg, cnt_inv, bases, comp, root, bias):
    n, d = x.shape
    msg = x[src]
    agg = jax.ops.segment_sum(msg, seg, num_segments=8 * n)
    agg = agg.reshape(8, n, d)
    w = jnp.einsum('rb,bio->rio', comp, bases)
    return _dense(agg, cnt_inv, x, w, root, bias.reshape(1, d))


def kernel(x, edge_index, edge_type, bases1, comp1, root1, bias1, gamma1, beta1,
           bases2, comp2, root2, bias2, gamma2, beta2, bases3, comp3, root3, bias3):
    n, d = x.shape
    src = edge_index[0]
    dst = edge_index[1]
    seg = edge_type * n + dst
    cnt = jax.ops.segment_sum(jnp.ones_like(src, jnp.float32), seg, num_segments=8 * n)
    inv = (1.0 / jnp.maximum(cnt, 1.0)).reshape(8, n, 1)

    h = _conv(x, src, seg, inv, bases1, comp1, root1, bias1)
    mu = jnp.mean(h, axis=0)
    var = jnp.var(h, axis=0)
    h = (h - mu) / jnp.sqrt(var + 1e-5) * gamma1 + beta1
    h = jax.nn.leaky_relu(h, negative_slope=0.1)

    h = _conv(h, src, seg, inv, bases2, comp2, root2, bias2)
    mu = jnp.mean(h, axis=0)
    var = jnp.var(h, axis=0)
    h = (h - mu) / jnp.sqrt(var + 1e-5) * gamma2 + beta2
    h = jax.nn.leaky_relu(h, negative_slope=0.1)

    h = _conv(h, src, seg, inv, bases3, comp3, root3, bias3)
    norm = jnp.linalg.norm(h, axis=1, keepdims=True)
    return h / jnp.maximum(norm, 1e-12)
